# baseline (speedup 1.0000x reference)
import functools

import jax
import jax.numpy as jnp
from jax import lax
from jax.experimental import pallas as pl
from jax.experimental.pallas import tpu as pltpu
from jax.experimental.pallas import tpu_sc as plsc

D = 384
K = 8192
BK = 8192
CCW = (0.05, 0.25, 0.6)

NC = 2
NS = 16
NW = NC * NS

_DOT_DTYPE = jnp.bfloat16


BN = 1024
NTOT = 9216
NRB = NTOT // BN


def _lvl_of(r):
    return (r + 3) // 4


def _argmin_body(ids_ref, zb2_ref, znorm_ref, cbnorm_ref,
                 cb0_ref, cb1_ref, cb2_ref, idx_ref, part_ref, cbv, sem):
    r = pl.program_id(0)

    def swap(cb_ref):
        c = pltpu.make_async_copy(cb_ref, cbv, sem)
        c.start()
        c.wait()

    @pl.when(r == 0)
    def _():
        swap(cb0_ref)

    @pl.when(r == 1)
    def _():
        swap(cb1_ref)

    @pl.when(r == 5)
    def _():
        swap(cb2_ref)

    cbb = cbv[...].astype(_DOT_DTYPE)
    m2 = lax.dot_general(zb2_ref[...], cbb, (((1,), (1,)), ((), ())),
                         preferred_element_type=jnp.float32)
    dist = (znorm_ref[...] + m2) + cbnorm_ref[0]
    m = jnp.min(dist, axis=1, keepdims=True)
    loc = jnp.min(jnp.where(dist == m, ids_ref[...], K),
                  axis=1, keepdims=True)
    idx_ref[...] = loc
    part_ref[...] = jnp.sum(m, keepdims=True)[None]


def _argmin_call(ids, zb2, znorm, cb0, cb1, cb2, cbnorm3, interpret=False):
    return pl.pallas_call(
        _argmin_body,
        grid=(NRB,),
        in_specs=[
            pl.BlockSpec((1, K), lambda r: (0, 0)),
            pl.BlockSpec((BN, D), lambda r: (r, 0)),
            pl.BlockSpec((BN, 1), lambda r: (r, 0)),
            pl.BlockSpec((1, 1, K), lambda r: (_lvl_of(r), 0, 0)),
            pl.BlockSpec(memory_space=pl.ANY),
            pl.BlockSpec(memory_space=pl.ANY),
            pl.BlockSpec(memory_space=pl.ANY),
        ],
        out_specs=[
            pl.BlockSpec((BN, 1), lambda r: (r, 0)),
            pl.BlockSpec((1, 1, 1), lambda r: (r, 0, 0)),
        ],
        out_shape=[
            jax.ShapeDtypeStruct((NTOT, 1), jnp.int32),
            jax.ShapeDtypeStruct((NRB, 1, 1), jnp.float32),
        ],
        scratch_shapes=[
            pltpu.VMEM((K, D), jnp.float32),
            pltpu.SemaphoreType.DMA,
        ],
        interpret=interpret,
    )(ids, zb2, znorm, cbnorm3, cb0, cb1, cb2)


@functools.lru_cache(maxsize=None)
def _make_gather(n):
    b_per_w = n // NW
    mesh = plsc.VectorSubcoreMesh(core_axis_name="c", subcore_axis_name="s")

    @functools.partial(
        pl.kernel,
        mesh=mesh,
        out_type=jax.ShapeDtypeStruct((n, D), jnp.float32),
        scratch_types=[
            pltpu.VMEM((b_per_w,), jnp.int32),
            pltpu.VMEM((b_per_w, D), jnp.float32),
            pltpu.SemaphoreType.DMA,
        ],
    )
    def gather(cb_hbm, idx_hbm, out_hbm, idx_v, rows_v, sem):
        wid = lax.axis_index("s") * NC + lax.axis_index("c")
        base = wid * b_per_w
        pltpu.sync_copy(idx_hbm.at[pl.ds(base, b_per_w)], idx_v)
        pltpu.async_copy(cb_hbm.at[idx_v], rows_v, sem).wait()
        pltpu.sync_copy(rows_v, out_hbm.at[pl.ds(base, b_per_w)])

    return gather


def kernel(l0, l1, l2, cb0, cb1, cb2):
    ids = jnp.arange(K, dtype=jnp.int32)[None, :]
    f0 = l0.reshape(-1, D)
    f1 = l1.reshape(-1, D)
    f2 = l2.reshape(-1, D)
    flat = jnp.concatenate([f0, f1, f2], axis=0)
    znorm = jnp.concatenate(
        [jnp.sum(f0 ** 2, axis=1, keepdims=True),
         jnp.sum(f1 ** 2, axis=1, keepdims=True),
         jnp.sum(f2 ** 2, axis=1, keepdims=True)], axis=0)
    cbnorm3 = jnp.stack(
        [jnp.sum(cb0 ** 2, axis=1)[None, :],
         jnp.sum(cb1 ** 2, axis=1)[None, :],
         jnp.sum(cb2 ** 2, axis=1)[None, :]], axis=0)
    zb2 = (-2.0 * flat).astype(_DOT_DTYPE)
    idx2d, part = _argmin_call(ids, zb2, znorm, cb0, cb1, cb2, cbnorm3)

    n0, n1 = 1024, 4096
    i0, i1, i2 = (idx2d[:n0], idx2d[n0:n0 + n1], idx2d[n0 + n1:])
    q0 = _make_gather(n0)(cb0, i0.reshape(-1)).reshape(l0.shape)
    q1 = _make_gather(n1)(cb1, i1.reshape(-1)).reshape(l1.shape)
    q2 = _make_gather(n1)(cb2, i2.reshape(-1)).reshape(l2.shape)
    nd0, nd1 = jnp.float32(n0 * D), jnp.float32(n1 * D)
    loss0 = jnp.float32(CCW[0]) * (jnp.sum(part[0:1]) / nd0)
    loss1 = jnp.float32(CCW[1]) * (jnp.sum(part[1:5]) / nd1)
    loss2 = jnp.float32(CCW[2]) * (jnp.sum(part[5:9]) / nd1)
    total = loss0 + loss1 + loss2
    return (i0.reshape(l0.shape[:-1]), i1.reshape(l1.shape[:-1]),
            i2.reshape(l2.shape[:-1]), total, q0, q1, q2)


# speedup vs baseline: 1.0247x; 1.0247x over previous
import functools

import jax
import jax.numpy as jnp
from jax import lax
from jax.experimental import pallas as pl
from jax.experimental.pallas import tpu as pltpu
from jax.experimental.pallas import tpu_sc as plsc

D = 384
K = 8192
BK = 8192
CCW = (0.05, 0.25, 0.6)

NC = 2
NS = 16
NW = NC * NS

_DOT_DTYPE = jnp.bfloat16


CK = 1024
NCK = K // CK


def _argmin_body(ids_ref, zb2_ref, znorm_ref, cb_ref, cbnorm_ref,
                 idx_ref, part_ref):
    zb2 = zb2_ref[...]
    zn = znorm_ref[...]
    m = None
    loc = None
    for c in range(NCK):
        cbb = cb_ref[c * CK:(c + 1) * CK, :].astype(_DOT_DTYPE)
        m2 = lax.dot_general(zb2, cbb, (((1,), (1,)), ((), ())),
                             preferred_element_type=jnp.float32)
        dist = (zn + m2) + cbnorm_ref[:, c * CK:(c + 1) * CK]
        mc = jnp.min(dist, axis=1, keepdims=True)
        locc = jnp.min(jnp.where(dist == mc, ids_ref[:, c * CK:(c + 1) * CK],
                                 K), axis=1, keepdims=True)
        if c == 0:
            m, loc = mc, locc
        else:
            better = mc < m
            m = jnp.where(better, mc, m)
            loc = jnp.where(better, locc, loc)
    idx_ref[...] = loc
    part_ref[...] = jnp.sum(m, keepdims=True)[None]


def _argmin_call(ids, zb2, znorm, cb, cbnorm, interpret=False):
    n = zb2.shape[0]
    bn = min(n, 1024)
    nrb = n // bn
    return pl.pallas_call(
        _argmin_body,
        grid=(nrb,),
        in_specs=[
            pl.BlockSpec((1, K), lambda r: (0, 0)),
            pl.BlockSpec((bn, D), lambda r: (r, 0)),
            pl.BlockSpec((bn, 1), lambda r: (r, 0)),
            pl.BlockSpec((K, D), lambda r: (0, 0)),
            pl.BlockSpec((1, K), lambda r: (0, 0)),
        ],
        out_specs=[
            pl.BlockSpec((bn, 1), lambda r: (r, 0)),
            pl.BlockSpec((1, 1, 1), lambda r: (r, 0, 0)),
        ],
        out_shape=[
            jax.ShapeDtypeStruct((n, 1), jnp.int32),
            jax.ShapeDtypeStruct((nrb, 1, 1), jnp.float32),
        ],
        interpret=interpret,
    )(ids, zb2, znorm, cb, cbnorm)


@functools.lru_cache(maxsize=None)
def _make_gather(n):
    b_per_w = n // NW
    mesh = plsc.VectorSubcoreMesh(core_axis_name="c", subcore_axis_name="s")

    @functools.partial(
        pl.kernel,
        mesh=mesh,
        out_type=jax.ShapeDtypeStruct((n, D), jnp.float32),
        scratch_types=[
            pltpu.VMEM((b_per_w,), jnp.int32),
            pltpu.VMEM((b_per_w, D), jnp.float32),
            pltpu.SemaphoreType.DMA,
        ],
    )
    def gather(cb_hbm, idx_hbm, out_hbm, idx_v, rows_v, sem):
        wid = lax.axis_index("s") * NC + lax.axis_index("c")
        base = wid * b_per_w
        pltpu.sync_copy(idx_hbm.at[pl.ds(base, b_per_w)], idx_v)
        pltpu.async_copy(cb_hbm.at[idx_v], rows_v, sem).wait()
        pltpu.sync_copy(rows_v, out_hbm.at[pl.ds(base, b_per_w)])

    return gather


def kernel(l0, l1, l2, cb0, cb1, cb2):
    ids = jnp.arange(K, dtype=jnp.int32)[None, :]
    out = []
    for i, (z, cb) in enumerate(((l0, cb0), (l1, cb1), (l2, cb2))):
        flat = z.reshape(-1, D)
        n = flat.shape[0]
        znorm = jnp.sum(flat ** 2, axis=1, keepdims=True)
        cbnorm = jnp.sum(cb ** 2, axis=1)[None, :]
        zb2 = (-2.0 * flat).astype(_DOT_DTYPE)
        idx2d, part = _argmin_call(ids, zb2, znorm, cb, cbnorm)
        idx = idx2d.reshape(z.shape[:-1])
        q = _make_gather(n)(cb, idx2d.reshape(-1)).reshape(z.shape)
        loss = jnp.float32(CCW[i]) * (jnp.sum(part) / jnp.float32(n * D))
        out.append((idx, loss, q))
    (idx0, loss0, q0), (idx1, loss1, q1), (idx2_, loss2, q2) = out
    total = loss0 + loss1 + loss2
    return (idx0, idx1, idx2_, total, q0, q1, q2)


# speedup vs baseline: 1.0365x; 1.0115x over previous
import functools

import jax
import jax.numpy as jnp
from jax import lax
from jax.experimental import pallas as pl
from jax.experimental.pallas import tpu as pltpu
from jax.experimental.pallas import tpu_sc as plsc

D = 384
K = 8192
BK = 8192
CCW = (0.05, 0.25, 0.6)

NC = 2
NS = 16
NW = NC * NS

_DOT_DTYPE = jnp.bfloat16


def _argmin_body(ids_ref, zb2_ref, znorm_ref, cb_ref, cbnorm_ref,
                 idx_ref, part_ref):
    cbb = cb_ref[...].astype(_DOT_DTYPE)
    m2 = lax.dot_general(zb2_ref[...], cbb, (((1,), (1,)), ((), ())),
                         preferred_element_type=jnp.float32)
    dist = (znorm_ref[...] + m2) + cbnorm_ref[...]
    m = jnp.min(dist, axis=1, keepdims=True)
    loc = jnp.min(jnp.where(dist == m, ids_ref[...], K),
                  axis=1, keepdims=True)
    idx_ref[...] = loc
    part_ref[...] = jnp.sum(m, keepdims=True)[None]


def _argmin_call(ids, zb2, znorm, cb, cbnorm, interpret=False):
    n = zb2.shape[0]
    bn = min(n, 1024)
    nrb = n // bn
    return pl.pallas_call(
        _argmin_body,
        grid=(nrb,),
        in_specs=[
            pl.BlockSpec((1, K), lambda r: (0, 0)),
            pl.BlockSpec((bn, D), lambda r: (r, 0)),
            pl.BlockSpec((bn, 1), lambda r: (r, 0)),
            pl.BlockSpec((K, D), lambda r: (0, 0)),
            pl.BlockSpec((1, K), lambda r: (0, 0)),
        ],
        out_specs=[
            pl.BlockSpec((bn, 1), lambda r: (r, 0)),
            pl.BlockSpec((1, 1, 1), lambda r: (r, 0, 0)),
        ],
        out_shape=[
            jax.ShapeDtypeStruct((n, 1), jnp.int32),
            jax.ShapeDtypeStruct((nrb, 1, 1), jnp.float32),
        ],
        interpret=interpret,
    )(ids, zb2, znorm, cb, cbnorm)


@functools.lru_cache(maxsize=None)
def _make_gather(n):
    b_per_w = n // NW
    mesh = plsc.VectorSubcoreMesh(core_axis_name="c", subcore_axis_name="s")

    @functools.partial(
        pl.kernel,
        mesh=mesh,
        out_type=jax.ShapeDtypeStruct((n, D), jnp.float32),
        scratch_types=[
            pltpu.VMEM((b_per_w,), jnp.int32),
            pltpu.VMEM((b_per_w, D), jnp.float32),
            pltpu.SemaphoreType.DMA,
        ],
    )
    def gather(cb_hbm, idx_hbm, out_hbm, idx_v, rows_v, sem):
        wid = lax.axis_index("s") * NC + lax.axis_index("c")
        base = wid * b_per_w
        pltpu.sync_copy(idx_hbm.at[pl.ds(base, b_per_w)], idx_v)
        pltpu.async_copy(cb_hbm.at[idx_v], rows_v, sem).wait()
        pltpu.sync_copy(rows_v, out_hbm.at[pl.ds(base, b_per_w)])

    return gather


def kernel(l0, l1, l2, cb0, cb1, cb2):
    ids = jnp.arange(K, dtype=jnp.int32)[None, :]
    out = []
    for i, (z, cb) in enumerate(((l0, cb0), (l1, cb1), (l2, cb2))):
        flat = z.reshape(-1, D)
        n = flat.shape[0]
        znorm = jnp.sum(flat ** 2, axis=1, keepdims=True)
        cbnorm = jnp.sum(cb ** 2, axis=1)[None, :]
        zb2 = (-2.0 * flat).astype(_DOT_DTYPE)
        idx2d, part = _argmin_call(ids, zb2, znorm, cb, cbnorm)
        idx = idx2d.reshape(z.shape[:-1])
        q = _make_gather(n)(cb, idx2d.reshape(-1)).reshape(z.shape)
        loss = jnp.float32(CCW[i]) * (jnp.sum(part) / jnp.float32(n * D))
        out.append((idx, loss, q))
    (idx0, loss0, q0), (idx1, loss1, q1), (idx2_, loss2, q2) = out
    total = loss0 + loss1 + loss2
    return (idx0, idx1, idx2_, total, q0, q1, q2)
